# Initial kernel scaffold; baseline (speedup 1.0000x reference)
import jax
import jax.numpy as jnp
from jax.experimental import pallas as pl


def kernel(vn_feat, vn_xyz, target_xyz, R_align, W1, b1, W2, b2):
    raise NotImplementedError("write your pallas kernel here")


# speedup vs baseline: 1730.2397x; 1730.2397x over previous
import functools

import jax
import jax.numpy as jnp
from jax import lax
from jax.experimental import pallas as pl
from jax.experimental.pallas import tpu as pltpu
from jax.experimental.pallas import tpu_sc as plsc

_B, _C, _M, _N, _K = 8, 256, 1024, 2048, 3
_C3 = 3 * _C
_HID = (3 * _C) // 2
_OUT = 128
_BN = _B * _N


def _a1_body(vf_ref, r_ref, fc_ref):
    for j in range(3):
        fc_ref[j] = (r_ref[0, 0:1, j:j + 1] * vf_ref[0]
                     + r_ref[0, 1:2, j:j + 1] * vf_ref[1]
                     + r_ref[0, 2:3, j:j + 1] * vf_ref[2])


def _stage_a1(vf_t, r_align):
    return pl.pallas_call(
        _a1_body,
        grid=(_B,),
        in_specs=[
            pl.BlockSpec((3, _C, _M), lambda b: (b, 0, 0)),
            pl.BlockSpec((1, 3, 3), lambda b: (b, 0, 0)),
        ],
        out_specs=pl.BlockSpec((3, _C, _M), lambda b: (b, 0, 0)),
        out_shape=jax.ShapeDtypeStruct((_B * 3, _C, _M), jnp.float32),
    )(vf_t, r_align)


def _a2_body(ft_ref, r_ref, w10_ref, w11_ref, w12_ref, vx_ref,
             proj_ref, vc_ref):
    ft = ft_ref[0]
    proj_ref[0] = (
        jnp.dot(ft[0:_M], w10_ref[...], preferred_element_type=jnp.float32)
        + jnp.dot(ft[_M:2 * _M], w11_ref[...],
                  preferred_element_type=jnp.float32)
        + jnp.dot(ft[2 * _M:3 * _M], w12_ref[...],
                  preferred_element_type=jnp.float32))
    def _bf(x):
        return x.astype(jnp.bfloat16).astype(jnp.float32)

    for j in range(3):
        vc_ref[0, j:j + 1, :] = (
            _bf(r_ref[0, 0:1, j:j + 1]) * _bf(vx_ref[0, 0:1, :])
            + _bf(r_ref[0, 1:2, j:j + 1]) * _bf(vx_ref[0, 1:2, :])
            + _bf(r_ref[0, 2:3, j:j + 1]) * _bf(vx_ref[0, 2:3, :]))


def _stage_a2(ft, r_align, w10, w11, w12, vn_xyz):
    return pl.pallas_call(
        _a2_body,
        grid=(_B,),
        in_specs=[
            pl.BlockSpec((1, 3 * _M, _C), lambda b: (b, 0, 0)),
            pl.BlockSpec((1, 3, 3), lambda b: (b, 0, 0)),
            pl.BlockSpec((_C, _HID), lambda b: (0, 0)),
            pl.BlockSpec((_C, _HID), lambda b: (0, 0)),
            pl.BlockSpec((_C, _HID), lambda b: (0, 0)),
            pl.BlockSpec((1, 3, _M), lambda b: (b, 0, 0)),
        ],
        out_specs=[
            pl.BlockSpec((1, _M, _HID), lambda b: (b, 0, 0)),
            pl.BlockSpec((1, 3, _M), lambda b: (b, 0, 0)),
        ],
        out_shape=[
            jax.ShapeDtypeStruct((_B, _M, _HID), jnp.float32),
            jax.ShapeDtypeStruct((_B, 3, _M), jnp.float32),
        ],
    )(ft, r_align, w10, w11, w12, vn_xyz)


_TN = 256


def _stage_b_body(tt_ref, vc_ref, idx_ref, w_ref):
    b = pl.program_id(0)
    t = tt_ref[0]
    v = vc_ref[0]
    d2 = None
    for i in range(3):
        diff = t[:, i:i + 1] - v[i:i + 1, :]
        sq = diff * diff
        d2 = sq if d2 is None else d2 + sq
    iota = lax.broadcasted_iota(jnp.int32, (_TN, _M), 1)
    inf = jnp.float32(jnp.inf)
    cur = d2
    vals, idxs = [], []
    for k in range(_K):
        mv = jnp.min(cur, axis=1, keepdims=True)
        mi = jnp.min(jnp.where(cur == mv, iota, _M), axis=1,
                     keepdims=True)
        vals.append(mv)
        idxs.append(mi)
        if k < _K - 1:
            cur = jnp.where(iota == mi, inf, cur)
    inv = [1.0 / (jnp.sqrt(jnp.maximum(vv, 0.0)) + 1e-8) for vv in vals]
    s = inv[0] + inv[1] + inv[2]
    w_ref[0] = jnp.concatenate(
        [jnp.broadcast_to(ik / s, (_TN, _L)) for ik in inv], axis=1)
    idx_ref[0] = jnp.concatenate(idxs, axis=1) + b * _M


def _stage_b(target_t, vc):
    return pl.pallas_call(
        _stage_b_body,
        grid=(_B, _N // _TN),
        in_specs=[
            pl.BlockSpec((1, _TN, 3), lambda b, n: (b, n, 0)),
            pl.BlockSpec((1, 3, _M), lambda b, n: (b, 0, 0)),
        ],
        out_specs=[
            pl.BlockSpec((1, _TN, 3), lambda b, n: (b, n, 0)),
            pl.BlockSpec((1, _TN, 3 * _L), lambda b, n: (b, n, 0)),
        ],
        out_shape=[
            jax.ShapeDtypeStruct((_B, _N, 3), jnp.int32),
            jax.ShapeDtypeStruct((_B, _N, 3 * _L), jnp.float32),
        ],
    )(target_t, vc)


_NC, _NS, _L = 2, 16, 16
_NW = _NC * _NS
_TPW = _BN // _NW
_CH = 16
_NCH = _TPW // _CH


def _stage_c_body(proj_hbm, idx_hbm, w_hbm, h_hbm, idx_v, w_v, rows_v, out_v,
                  sem):
    wid = lax.axis_index("s") * _NC + lax.axis_index("c")
    base = wid * _TPW

    def chunk(ch, carry):
        t0 = base + ch * _CH
        pltpu.sync_copy(idx_hbm.at[pl.ds(t0 * 3, _CH * 3)], idx_v)
        pltpu.sync_copy(w_hbm.at[pl.ds(t0 * 3 * _L, _CH * 3 * _L)], w_v)
        pltpu.async_copy(proj_hbm.at[idx_v], rows_v, sem).wait()
        for t in range(_CH):
            w0 = w_v[pl.ds(t * 3 * _L, _L)]
            w1 = w_v[pl.ds(t * 3 * _L + _L, _L)]
            w2 = w_v[pl.ds(t * 3 * _L + 2 * _L, _L)]
            for d in range(_HID // _L):
                sl = pl.ds(d * _L, _L)
                out_v[t, sl] = (rows_v[3 * t, sl] * w0
                                + rows_v[3 * t + 1, sl] * w1
                                + rows_v[3 * t + 2, sl] * w2)
        pltpu.sync_copy(out_v, h_hbm.at[pl.ds(t0, _CH)])
        return carry

    lax.fori_loop(0, _NCH, chunk, 0)


@functools.cache
def _make_stage_c():
    return pl.kernel(
        _stage_c_body,
        out_type=jax.ShapeDtypeStruct((_BN, _HID), jnp.float32),
        mesh=plsc.VectorSubcoreMesh(core_axis_name="c", subcore_axis_name="s"),
        scratch_types=[
            pltpu.VMEM((_CH * 3,), jnp.int32),
            pltpu.VMEM((_CH * 3 * _L,), jnp.float32),
            pltpu.VMEM((_CH * 3, _HID), jnp.float32),
            pltpu.VMEM((_CH, _HID), jnp.float32),
            pltpu.SemaphoreType.DMA,
        ],
    )


_TD = 2048


def _stage_d_body(h_ref, b1_ref, w2_ref, b2_ref, o_ref):
    x = jnp.maximum(h_ref[...] + b1_ref[...], 0.0)
    o_ref[...] = (jnp.dot(x, w2_ref[...], preferred_element_type=jnp.float32)
                  + b2_ref[...])


def _stage_d(h, b1, w2, b2):
    return pl.pallas_call(
        _stage_d_body,
        grid=(_BN // _TD,),
        in_specs=[
            pl.BlockSpec((_TD, _HID), lambda i: (i, 0)),
            pl.BlockSpec((1, _HID), lambda i: (0, 0)),
            pl.BlockSpec((_HID, _OUT), lambda i: (0, 0)),
            pl.BlockSpec((1, _OUT), lambda i: (0, 0)),
        ],
        out_specs=pl.BlockSpec((_TD, _OUT), lambda i: (i, 0)),
        out_shape=jax.ShapeDtypeStruct((_BN, _OUT), jnp.float32),
    )(h, b1, w2, b2)


def kernel(vn_feat, vn_xyz, target_xyz, R_align, W1, b1, W2, b2):
    vf_t = vn_feat.transpose(0, 3, 1, 2).reshape(_B * 3, _C, _M)
    w13 = W1.reshape(_C, 3, _HID)
    w10, w11, w12 = w13[:, 0, :], w13[:, 1, :], w13[:, 2, :]
    target_t = target_xyz.transpose(0, 2, 1)

    fct = _stage_a1(vf_t, R_align)
    ft = (fct.reshape(_B, 3, _C, _M).transpose(0, 3, 1, 2)
          .reshape(_B, 3 * _M, _C))
    proj, vc = _stage_a2(ft, R_align, w10, w11, w12, vn_xyz)
    idxg, w = _stage_b(target_t, vc)

    h = _make_stage_c()(proj.reshape(_B * _M, _HID),
                        idxg.reshape(_BN * 3),
                        w.reshape(_BN * 3 * _L))

    y = _stage_d(h, b1.reshape(1, _HID), W2, b2.reshape(1, _OUT))
    return y.reshape(_B, _N, _OUT).transpose(0, 2, 1)
